# baseline (speedup 1.0000x reference)
import functools

import jax
import jax.numpy as jnp
from jax import lax
from jax.experimental import pallas as pl
from jax.experimental.pallas import tpu as pltpu
from jax.experimental.pallas import tpu_sc as plsc

_NC = 2
_NS = 16
_NW = _NC * _NS
_K = 4


def _sigmoid_body(x_ref, o_ref):
    o_ref[...] = jax.nn.sigmoid(x_ref[...])


def _gather_body(table_hbm, idx_hbm, out_hbm, idx_v, buf_a, buf_b,
                 gsem_a, gsem_b, wsem_a, wsem_b):
    b_per_w = idx_v.shape[0]
    c = b_per_w // _K
    wid = lax.axis_index("s") * _NC + lax.axis_index("c")
    base = wid * b_per_w
    pltpu.sync_copy(idx_hbm.at[pl.ds(base, b_per_w)], idx_v)

    bufs = [buf_a, buf_b]
    gsems = [gsem_a, gsem_b]
    wsems = [wsem_a, wsem_b]

    def gather(i):
        return pltpu.async_copy(
            table_hbm.at[idx_v.at[pl.ds(i * c, c)]], bufs[i % 2], gsems[i % 2])

    def write(i):
        return pltpu.async_copy(
            bufs[i % 2], out_hbm.at[pl.ds(base + i * c, c)], wsems[i % 2])

    g = [None] * _K
    w = [None] * _K
    for i in range(_K):
        if i >= 2:
            w[i - 2].wait()
        g[i] = gather(i)
        if i >= 1:
            g[i - 1].wait()
            w[i - 1] = write(i - 1)
    g[_K - 1].wait()
    w[_K - 1] = write(_K - 1)
    if _K >= 2:
        w[_K - 2].wait()
    w[_K - 1].wait()


def kernel(mask, idx):
    i, d = mask.shape
    b = idx.shape[0]
    b_per_w = b // _NW

    table = pl.pallas_call(
        _sigmoid_body,
        out_shape=jax.ShapeDtypeStruct((i, d), jnp.float32),
    )(mask)

    mesh = plsc.VectorSubcoreMesh(core_axis_name="c", subcore_axis_name="s")
    gather = functools.partial(
        pl.kernel,
        mesh=mesh,
        out_type=jax.ShapeDtypeStruct((b, d), jnp.float32),
        scratch_types=[
            pltpu.VMEM((b_per_w,), jnp.int32),
            pltpu.VMEM((b_per_w // _K, d), jnp.float32),
            pltpu.VMEM((b_per_w // _K, d), jnp.float32),
            pltpu.SemaphoreType.DMA,
            pltpu.SemaphoreType.DMA,
            pltpu.SemaphoreType.DMA,
            pltpu.SemaphoreType.DMA,
        ],
    )(_gather_body)
    return gather(table, idx.astype(jnp.int32))
